# baseline (speedup 1.0000x reference)
import functools

import jax
import jax.numpy as jnp
from jax import lax
from jax.experimental import pallas as pl
from jax.experimental.pallas import tpu as pltpu
from jax.experimental.pallas import tpu_sc as plsc

_DEPTH = 22
_ATOMS = 5
_PLANES = _ATOMS * _DEPTH
_PLANE_ROWS = 2048
_ROWS = _PLANES * _PLANE_ROWS
_CROWS = 16
_NCHUNK = 64 // _CROWS
_NPAIR = _DEPTH // 2
_T = _NCHUNK * _NPAIR


def _sc_body(x4_hbm, o4_hbm, strips, best, idx, ohs, si0, si1, so0, so1):
    wid = lax.axis_index("s") * 2 + lax.axis_index("c")
    w64 = wid * 64

    def in_copy(t, bp, sem):
        c = t // _NPAIR
        d0 = 2 * (t % _NPAIR)
        return pltpu.make_async_copy(
            x4_hbm.at[:, pl.ds(d0, 2), pl.ds(w64 + c * _CROWS, _CROWS), :],
            strips.at[bp],
            sem,
        )

    def out_copies(t, sp, sem):
        c2 = t // _NPAIR - 1
        j2 = t % _NPAIR
        cps = []
        for dd in range(2):
            for a_ in range(_ATOMS):
                cps.append(
                    pltpu.make_async_copy(
                        ohs.at[sp, dd],
                        o4_hbm.at[
                            a_,
                            2 * j2 + dd,
                            pl.ds(w64 + c2 * _CROWS, _CROWS),
                            :,
                        ],
                        sem,
                    )
                )
        return cps

    def p1_compute(t, bp):
        j = t % _NPAIR
        cpar = (t // _NPAIR) & 1
        d0v = jnp.full((16,), 2 * j, jnp.int32)

        def sum5(dd, r, cc):
            return (
                strips[bp, 0, dd, r, pl.ds(cc, 16)]
                + strips[bp, 1, dd, r, pl.ds(cc, 16)]
                + strips[bp, 2, dd, r, pl.ds(cc, 16)]
                + strips[bp, 3, dd, r, pl.ds(cc, 16)]
                + strips[bp, 4, dd, r, pl.ds(cc, 16)]
            )

        @pl.when(j == 0)
        def _():
            def r_body(r, cr):
                for k in range(8):
                    cc = k * 16
                    s0 = sum5(0, r, cc)
                    s1 = sum5(1, r, cc)
                    gt1 = s1 > s0
                    best[r, pl.ds(cc, 16)] = jnp.where(gt1, s1, s0)
                    idx[cpar, r, pl.ds(cc, 16)] = jnp.where(gt1, d0v + 1, d0v)
                return cr

            lax.fori_loop(0, _CROWS, r_body, 0)

        @pl.when(j > 0)
        def _():
            def r_body(r, cr):
                for k in range(8):
                    cc = k * 16
                    s0 = sum5(0, r, cc)
                    s1 = sum5(1, r, cc)
                    gt1 = s1 > s0
                    sm = jnp.where(gt1, s1, s0)
                    dm = jnp.where(gt1, d0v + 1, d0v)
                    b = best[r, pl.ds(cc, 16)]
                    gt = sm > b
                    best[r, pl.ds(cc, 16)] = jnp.where(gt, sm, b)
                    iv = idx[cpar, r, pl.ds(cc, 16)]
                    idx[cpar, r, pl.ds(cc, 16)] = jnp.where(gt, dm, iv)
                return cr

            lax.fori_loop(0, _CROWS, r_body, 0)

    def p2_compute(t, sp):
        j2 = t % _NPAIR
        cpar = (t // _NPAIR - 1) & 1
        d0v = jnp.full((16,), 2 * j2, jnp.int32)
        one = jnp.full((16,), 1.0, jnp.float32)
        zero = jnp.full((16,), 0.0, jnp.float32)

        def r_body(r, cr):
            for k in range(8):
                cc = k * 16
                iv = idx[cpar, r, pl.ds(cc, 16)]
                ohs[sp, 0, r, pl.ds(cc, 16)] = jnp.where(iv == d0v, one, zero)
                ohs[sp, 1, r, pl.ds(cc, 16)] = jnp.where(
                    iv == d0v + 1, one, zero
                )
            return cr

        lax.fori_loop(0, _CROWS, r_body, 0)

    in_copy(0, 0, si0).start()
    in_copy(1, 1, si1).start()

    def step(t, c):
        @pl.when(t < _T)
        def _():
            @pl.when((t & 1) == 0)
            def _():
                in_copy(t, 0, si0).wait()
                p1_compute(t, 0)

            @pl.when((t & 1) == 1)
            def _():
                in_copy(t, 1, si1).wait()
                p1_compute(t, 1)

            @pl.when(t + 2 < _T)
            def _():
                @pl.when((t & 1) == 0)
                def _():
                    in_copy(t + 2, 0, si0).start()

                @pl.when((t & 1) == 1)
                def _():
                    in_copy(t + 2, 1, si1).start()

        @pl.when(t >= _NPAIR)
        def _():
            @pl.when((t & 1) == 0)
            def _():
                @pl.when(t >= _NPAIR + 2)
                def _():
                    for cp in out_copies(t - 2, 0, so0):
                        cp.wait()

                p2_compute(t, 0)
                for cp in out_copies(t, 0, so0):
                    cp.start()

            @pl.when((t & 1) == 1)
            def _():
                @pl.when(t >= _NPAIR + 2)
                def _():
                    for cp in out_copies(t - 2, 1, so1):
                        cp.wait()

                p2_compute(t, 1)
                for cp in out_copies(t, 1, so1):
                    cp.start()

        return c

    lax.fori_loop(0, _T + _NPAIR, step, 0)

    tlast = _T + _NPAIR
    for cp in out_copies(tlast - 2, (tlast - 2) & 1, [so0, so1][(tlast - 2) & 1]):
        cp.wait()
    for cp in out_copies(tlast - 1, (tlast - 1) & 1, [so0, so1][(tlast - 1) & 1]):
        cp.wait()


def kernel(inputs):
    b, l, a, d = inputs.shape
    x4 = (
        jnp.transpose(inputs, (2, 3, 0, 1))
        .reshape(_PLANES, b // 8, 8, l // 128, 128)
        .transpose(0, 1, 3, 2, 4)
        .reshape(a, d, _PLANE_ROWS, 128)
    )
    mesh = plsc.VectorSubcoreMesh(core_axis_name="c", subcore_axis_name="s")
    f = pl.kernel(
        _sc_body,
        out_type=jax.ShapeDtypeStruct((a, d, _PLANE_ROWS, 128), jnp.float32),
        mesh=mesh,
        scratch_types=[
            pltpu.VMEM((2, _ATOMS, 2, _CROWS, 128), jnp.float32),
            pltpu.VMEM((_CROWS, 128), jnp.float32),
            pltpu.VMEM((2, _CROWS, 128), jnp.int32),
            pltpu.VMEM((2, 2, _CROWS, 128), jnp.float32),
            pltpu.SemaphoreType.DMA,
            pltpu.SemaphoreType.DMA,
            pltpu.SemaphoreType.DMA,
            pltpu.SemaphoreType.DMA,
        ],
    )
    o4 = f(x4)
    return (
        o4.reshape(_PLANES, b // 8, l // 128, 8, 128)
        .transpose(0, 1, 3, 2, 4)
        .reshape(a, d, b, l)
        .transpose(2, 3, 0, 1)
    )


# speedup vs baseline: 1.3735x; 1.3735x over previous
import functools

import jax
import jax.numpy as jnp
from jax import lax
from jax.experimental import pallas as pl
from jax.experimental.pallas import tpu as pltpu
from jax.experimental.pallas import tpu_sc as plsc

_DEPTH = 22
_ATOMS = 5
_PLANES = _ATOMS * _DEPTH
_PLANE_ROWS = 2048
_ROWS = _PLANES * _PLANE_ROWS
_WROWS = 32


def _sc_body(x_hbm, o_hbm, strips, best, idx0, idx1, ohs, si0, si1, so0, so1):
    wid = lax.axis_index("s") * 2 + lax.axis_index("c")

    def in_pair(base, d0, bp, sem):
        cps = []
        for dd in range(2):
            for a_ in range(_ATOMS):
                cps.append(
                    pltpu.make_async_copy(
                        x_hbm.at[
                            pl.ds(
                                (a_ * _DEPTH + d0 + dd) * _PLANE_ROWS + base,
                                _WROWS,
                            ),
                            :,
                        ],
                        strips.at[2 * bp + dd, a_],
                        sem,
                    )
                )
        return cps

    def out_slot(base, j2, sp, sem):
        cps = []
        for dd in range(2):
            for a_ in range(_ATOMS):
                cps.append(
                    pltpu.make_async_copy(
                        ohs.at[sp, dd],
                        o_hbm.at[
                            pl.ds(
                                (a_ * _DEPTH + 2 * j2 + dd) * _PLANE_ROWS
                                + base,
                                _WROWS,
                            ),
                            :,
                        ],
                        sem,
                    )
                )
        return cps

    def start(cps):
        for cp in cps:
            cp.start()

    def wait(cps):
        for cp in cps:
            cp.wait()

    def sum5(bk, r, cc):
        return (
            strips[bk, 0, r, pl.ds(cc, 16)]
            + strips[bk, 1, r, pl.ds(cc, 16)]
            + strips[bk, 2, r, pl.ds(cc, 16)]
            + strips[bk, 3, r, pl.ds(cc, 16)]
            + strips[bk, 4, r, pl.ds(cc, 16)]
        )

    def compute_pair(idx, bp, d0, first):
        d0v = jnp.full((16,), d0, jnp.int32)

        def r_body(r, c):
            for k in range(8):
                cc = k * 16
                s0 = sum5(2 * bp, r, cc)
                s1 = sum5(2 * bp + 1, r, cc)
                gt1 = s1 > s0
                sm = jnp.where(gt1, s1, s0)
                dm = jnp.where(gt1, d0v + 1, d0v)
                if first:
                    best[r, pl.ds(cc, 16)] = sm
                    idx[r, pl.ds(cc, 16)] = dm
                else:
                    b = best[r, pl.ds(cc, 16)]
                    gt = sm > b
                    best[r, pl.ds(cc, 16)] = jnp.where(gt, sm, b)
                    iv = idx[r, pl.ds(cc, 16)]
                    idx[r, pl.ds(cc, 16)] = jnp.where(gt, dm, iv)
            return c

        lax.fori_loop(0, _WROWS, r_body, 0)

    def p2_build(idx, j2, sp):
        d0v = jnp.full((16,), 2 * j2, jnp.int32)
        one = jnp.full((16,), 1.0, jnp.float32)
        zero = jnp.full((16,), 0.0, jnp.float32)

        def r_body(r, c):
            for k in range(8):
                cc = k * 16
                iv = idx[r, pl.ds(cc, 16)]
                ohs[sp, 0, r, pl.ds(cc, 16)] = jnp.where(iv == d0v, one, zero)
                ohs[sp, 1, r, pl.ds(cc, 16)] = jnp.where(
                    iv == d0v + 1, one, zero
                )
            return c

        lax.fori_loop(0, _WROWS, r_body, 0)

    sos = [so0, so1]
    base0 = wid * 64
    base1 = base0 + _WROWS

    start(in_pair(base0, 0, 0, si0))
    start(in_pair(base0, 2, 1, si1))
    wait(in_pair(base0, 0, 0, si0))
    compute_pair(idx0, 0, 0, True)
    start(in_pair(base0, 4, 0, si0))
    wait(in_pair(base0, 2, 1, si1))
    compute_pair(idx0, 1, jnp.int32(2), False)
    start(in_pair(base0, 6, 1, si1))

    def a_body(i, c):
        d0 = 4 * i + 4
        wait(in_pair(base0, d0, 0, si0))
        compute_pair(idx0, 0, d0, False)
        start(in_pair(base0, d0 + 4, 0, si0))
        wait(in_pair(base0, d0 + 2, 1, si1))
        compute_pair(idx0, 1, d0 + 2, False)

        @pl.when(d0 + 6 < _DEPTH)
        def _():
            start(in_pair(base0, d0 + 6, 1, si1))

        return c

    lax.fori_loop(0, 4, a_body, 0)
    wait(in_pair(base0, _DEPTH - 2, 0, si0))
    compute_pair(idx0, 0, jnp.int32(_DEPTH - 2), False)

    start(in_pair(base1, 0, 0, si0))
    start(in_pair(base1, 2, 1, si1))
    wait(in_pair(base1, 0, 0, si0))
    compute_pair(idx1, 0, 0, True)
    start(in_pair(base1, 4, 0, si0))
    p2_build(idx0, jnp.int32(0), 0)
    start(out_slot(base0, jnp.int32(0), 0, so0))
    wait(in_pair(base1, 2, 1, si1))
    compute_pair(idx1, 1, jnp.int32(2), False)
    start(in_pair(base1, 6, 1, si1))
    p2_build(idx0, jnp.int32(1), 1)
    start(out_slot(base0, jnp.int32(1), 1, so1))

    def b_body(i, c):
        d0 = 4 * i + 4
        j2a = 2 * i + 2
        wait(in_pair(base1, d0, 0, si0))
        compute_pair(idx1, 0, d0, False)
        start(in_pair(base1, d0 + 4, 0, si0))
        wait(out_slot(base0, j2a - 2, 0, so0))
        p2_build(idx0, j2a, 0)
        start(out_slot(base0, j2a, 0, so0))

        wait(in_pair(base1, d0 + 2, 1, si1))
        compute_pair(idx1, 1, d0 + 2, False)

        @pl.when(d0 + 6 < _DEPTH)
        def _():
            start(in_pair(base1, d0 + 6, 1, si1))

        wait(out_slot(base0, j2a - 1, 1, so1))
        p2_build(idx0, j2a + 1, 1)
        start(out_slot(base0, j2a + 1, 1, so1))
        return c

    lax.fori_loop(0, 4, b_body, 0)
    wait(in_pair(base1, _DEPTH - 2, 0, si0))
    compute_pair(idx1, 0, jnp.int32(_DEPTH - 2), False)
    wait(out_slot(base0, jnp.int32(8), 0, so0))
    p2_build(idx0, jnp.int32(10), 0)
    start(out_slot(base0, jnp.int32(10), 0, so0))

    wait(out_slot(base0, jnp.int32(10), 0, so0))
    p2_build(idx1, jnp.int32(0), 0)
    start(out_slot(base1, jnp.int32(0), 0, so0))
    wait(out_slot(base0, jnp.int32(9), 1, so1))
    p2_build(idx1, jnp.int32(1), 1)
    start(out_slot(base1, jnp.int32(1), 1, so1))

    def c_body(i, c):
        j2 = 2 * i + 2
        wait(out_slot(base1, j2 - 2, 0, so0))
        p2_build(idx1, j2, 0)
        start(out_slot(base1, j2, 0, so0))
        wait(out_slot(base1, j2 - 1, 1, so1))
        p2_build(idx1, j2 + 1, 1)
        start(out_slot(base1, j2 + 1, 1, so1))
        return c

    lax.fori_loop(0, 4, c_body, 0)
    wait(out_slot(base1, jnp.int32(8), 0, so0))
    p2_build(idx1, jnp.int32(10), 0)
    start(out_slot(base1, jnp.int32(10), 0, so0))
    wait(out_slot(base1, jnp.int32(9), 1, so1))
    wait(out_slot(base1, jnp.int32(10), 0, so0))


def kernel(inputs):
    b, l, a, d = inputs.shape
    x2 = (
        jnp.transpose(inputs, (2, 3, 0, 1))
        .reshape(_PLANES, b // 8, 8, l // 128, 128)
        .transpose(0, 1, 3, 2, 4)
        .reshape(_ROWS, 128)
    )
    mesh = plsc.VectorSubcoreMesh(core_axis_name="c", subcore_axis_name="s")
    f = pl.kernel(
        _sc_body,
        out_type=jax.ShapeDtypeStruct((_ROWS, 128), jnp.float32),
        mesh=mesh,
        scratch_types=[
            pltpu.VMEM((4, _ATOMS, _WROWS, 128), jnp.float32),
            pltpu.VMEM((_WROWS, 128), jnp.float32),
            pltpu.VMEM((_WROWS, 128), jnp.int32),
            pltpu.VMEM((_WROWS, 128), jnp.int32),
            pltpu.VMEM((2, 2, _WROWS, 128), jnp.float32),
            pltpu.SemaphoreType.DMA,
            pltpu.SemaphoreType.DMA,
            pltpu.SemaphoreType.DMA,
            pltpu.SemaphoreType.DMA,
        ],
    )
    o2 = f(x2)
    return (
        o2.reshape(_PLANES, b // 8, l // 128, 8, 128)
        .transpose(0, 1, 3, 2, 4)
        .reshape(a, d, b, l)
        .transpose(2, 3, 0, 1)
    )


# speedup vs baseline: 1.3835x; 1.0073x over previous
import functools

import jax
import jax.numpy as jnp
from jax import lax
from jax.experimental import pallas as pl
from jax.experimental.pallas import tpu as pltpu
from jax.experimental.pallas import tpu_sc as plsc

_DEPTH = 22
_ATOMS = 5
_PLANES = _ATOMS * _DEPTH
_PLANE_ROWS = 2048
_ROWS = _PLANES * _PLANE_ROWS
_WROWS = 32


def _sc_body(x4_hbm, o_hbm, strips, best, idx0, idx1, ohs, si0, si1, so0, so1):
    wid = lax.axis_index("s") * 2 + lax.axis_index("c")

    def in_pair(base, d0, bp, sem):
        return [
            pltpu.make_async_copy(
                x4_hbm.at[:, pl.ds(d0, 2), pl.ds(base, _WROWS), :],
                strips.at[bp],
                sem,
            )
        ]

    def out_slot(base, j2, sp, sem):
        cps = []
        for dd in range(2):
            for a_ in range(_ATOMS):
                cps.append(
                    pltpu.make_async_copy(
                        ohs.at[sp, dd],
                        o_hbm.at[
                            pl.ds(
                                (a_ * _DEPTH + 2 * j2 + dd) * _PLANE_ROWS
                                + base,
                                _WROWS,
                            ),
                            :,
                        ],
                        sem,
                    )
                )
        return cps

    def start(cps):
        for cp in cps:
            cp.start()

    def wait(cps):
        for cp in cps:
            cp.wait()

    def sum5(bp, dd, r, cc):
        return (
            strips[bp, 0, dd, r, pl.ds(cc, 16)]
            + strips[bp, 1, dd, r, pl.ds(cc, 16)]
            + strips[bp, 2, dd, r, pl.ds(cc, 16)]
            + strips[bp, 3, dd, r, pl.ds(cc, 16)]
            + strips[bp, 4, dd, r, pl.ds(cc, 16)]
        )

    def compute_pair(idx, bp, d0, first):
        d0v = jnp.full((16,), d0, jnp.int32)

        def r_body(r, c):
            for k in range(8):
                cc = k * 16
                s0 = sum5(bp, 0, r, cc)
                s1 = sum5(bp, 1, r, cc)
                gt1 = s1 > s0
                sm = jnp.where(gt1, s1, s0)
                dm = jnp.where(gt1, d0v + 1, d0v)
                if first:
                    best[r, pl.ds(cc, 16)] = sm
                    idx[r, pl.ds(cc, 16)] = dm
                else:
                    b = best[r, pl.ds(cc, 16)]
                    gt = sm > b
                    best[r, pl.ds(cc, 16)] = jnp.where(gt, sm, b)
                    iv = idx[r, pl.ds(cc, 16)]
                    idx[r, pl.ds(cc, 16)] = jnp.where(gt, dm, iv)
            return c

        lax.fori_loop(0, _WROWS, r_body, 0)

    def p2_build(idx, j2, sp):
        d0v = jnp.full((16,), 2 * j2, jnp.int32)
        one = jnp.full((16,), 1.0, jnp.float32)
        zero = jnp.full((16,), 0.0, jnp.float32)

        def r_body(r, c):
            for k in range(8):
                cc = k * 16
                iv = idx[r, pl.ds(cc, 16)]
                ohs[sp, 0, r, pl.ds(cc, 16)] = jnp.where(iv == d0v, one, zero)
                ohs[sp, 1, r, pl.ds(cc, 16)] = jnp.where(
                    iv == d0v + 1, one, zero
                )
            return c

        lax.fori_loop(0, _WROWS, r_body, 0)

    sos = [so0, so1]
    base0 = wid * 64
    base1 = base0 + _WROWS

    start(in_pair(base0, 0, 0, si0))
    start(in_pair(base0, 2, 1, si1))
    wait(in_pair(base0, 0, 0, si0))
    compute_pair(idx0, 0, 0, True)
    start(in_pair(base0, 4, 0, si0))
    wait(in_pair(base0, 2, 1, si1))
    compute_pair(idx0, 1, jnp.int32(2), False)
    start(in_pair(base0, 6, 1, si1))

    def a_body(i, c):
        d0 = 4 * i + 4
        wait(in_pair(base0, d0, 0, si0))
        compute_pair(idx0, 0, d0, False)
        start(in_pair(base0, d0 + 4, 0, si0))
        wait(in_pair(base0, d0 + 2, 1, si1))
        compute_pair(idx0, 1, d0 + 2, False)

        @pl.when(d0 + 6 < _DEPTH)
        def _():
            start(in_pair(base0, d0 + 6, 1, si1))

        return c

    lax.fori_loop(0, 4, a_body, 0)
    wait(in_pair(base0, _DEPTH - 2, 0, si0))
    compute_pair(idx0, 0, jnp.int32(_DEPTH - 2), False)

    start(in_pair(base1, 0, 0, si0))
    start(in_pair(base1, 2, 1, si1))
    wait(in_pair(base1, 0, 0, si0))
    compute_pair(idx1, 0, 0, True)
    start(in_pair(base1, 4, 0, si0))
    p2_build(idx0, jnp.int32(0), 0)
    start(out_slot(base0, jnp.int32(0), 0, so0))
    wait(in_pair(base1, 2, 1, si1))
    compute_pair(idx1, 1, jnp.int32(2), False)
    start(in_pair(base1, 6, 1, si1))
    p2_build(idx0, jnp.int32(1), 1)
    start(out_slot(base0, jnp.int32(1), 1, so1))

    def b_body(i, c):
        d0 = 4 * i + 4
        j2a = 2 * i + 2
        wait(in_pair(base1, d0, 0, si0))
        compute_pair(idx1, 0, d0, False)
        start(in_pair(base1, d0 + 4, 0, si0))
        wait(out_slot(base0, j2a - 2, 0, so0))
        p2_build(idx0, j2a, 0)
        start(out_slot(base0, j2a, 0, so0))

        wait(in_pair(base1, d0 + 2, 1, si1))
        compute_pair(idx1, 1, d0 + 2, False)

        @pl.when(d0 + 6 < _DEPTH)
        def _():
            start(in_pair(base1, d0 + 6, 1, si1))

        wait(out_slot(base0, j2a - 1, 1, so1))
        p2_build(idx0, j2a + 1, 1)
        start(out_slot(base0, j2a + 1, 1, so1))
        return c

    lax.fori_loop(0, 4, b_body, 0)
    wait(in_pair(base1, _DEPTH - 2, 0, si0))
    compute_pair(idx1, 0, jnp.int32(_DEPTH - 2), False)
    wait(out_slot(base0, jnp.int32(8), 0, so0))
    p2_build(idx0, jnp.int32(10), 0)
    start(out_slot(base0, jnp.int32(10), 0, so0))

    wait(out_slot(base0, jnp.int32(10), 0, so0))
    p2_build(idx1, jnp.int32(0), 0)
    start(out_slot(base1, jnp.int32(0), 0, so0))
    wait(out_slot(base0, jnp.int32(9), 1, so1))
    p2_build(idx1, jnp.int32(1), 1)
    start(out_slot(base1, jnp.int32(1), 1, so1))

    def c_body(i, c):
        j2 = 2 * i + 2
        wait(out_slot(base1, j2 - 2, 0, so0))
        p2_build(idx1, j2, 0)
        start(out_slot(base1, j2, 0, so0))
        wait(out_slot(base1, j2 - 1, 1, so1))
        p2_build(idx1, j2 + 1, 1)
        start(out_slot(base1, j2 + 1, 1, so1))
        return c

    lax.fori_loop(0, 4, c_body, 0)
    wait(out_slot(base1, jnp.int32(8), 0, so0))
    p2_build(idx1, jnp.int32(10), 0)
    start(out_slot(base1, jnp.int32(10), 0, so0))
    wait(out_slot(base1, jnp.int32(9), 1, so1))
    wait(out_slot(base1, jnp.int32(10), 0, so0))


def kernel(inputs):
    b, l, a, d = inputs.shape
    x4 = (
        jnp.transpose(inputs, (2, 3, 0, 1))
        .reshape(_PLANES, b // 8, 8, l // 128, 128)
        .transpose(0, 1, 3, 2, 4)
        .reshape(a, d, _PLANE_ROWS, 128)
    )
    mesh = plsc.VectorSubcoreMesh(core_axis_name="c", subcore_axis_name="s")
    f = pl.kernel(
        _sc_body,
        out_type=jax.ShapeDtypeStruct((_ROWS, 128), jnp.float32),
        mesh=mesh,
        scratch_types=[
            pltpu.VMEM((2, _ATOMS, 2, _WROWS, 128), jnp.float32),
            pltpu.VMEM((_WROWS, 128), jnp.float32),
            pltpu.VMEM((_WROWS, 128), jnp.int32),
            pltpu.VMEM((_WROWS, 128), jnp.int32),
            pltpu.VMEM((2, 2, _WROWS, 128), jnp.float32),
            pltpu.SemaphoreType.DMA,
            pltpu.SemaphoreType.DMA,
            pltpu.SemaphoreType.DMA,
            pltpu.SemaphoreType.DMA,
        ],
    )
    o2 = f(x4)
    return (
        o2.reshape(_PLANES, b // 8, l // 128, 8, 128)
        .transpose(0, 1, 3, 2, 4)
        .reshape(a, d, b, l)
        .transpose(2, 3, 0, 1)
    )


# speedup vs baseline: 1.4283x; 1.0323x over previous
import functools

import jax
import jax.numpy as jnp
from jax import lax
from jax.experimental import pallas as pl
from jax.experimental.pallas import tpu as pltpu
from jax.experimental.pallas import tpu_sc as plsc

_DEPTH = 22
_ATOMS = 5
_PLANES = _ATOMS * _DEPTH
_PLANE_ROWS = 2048
_ROWS = _PLANES * _PLANE_ROWS
_WROWS = 32


def _sc_body(x4_hbm, o_hbm, strips, best, idx0, idx1, ohs, si0, si1, so0, so1):
    wid = lax.axis_index("s") * 2 + lax.axis_index("c")

    def in_pair(base, d0, bp, sem):
        return [
            pltpu.make_async_copy(
                x4_hbm.at[:, pl.ds(d0, 2), pl.ds(base, _WROWS), :],
                strips.at[bp],
                sem,
            )
        ]

    def out_slot(base, j2, sp, sem):
        cps = []
        for dd in range(2):
            for a_ in range(_ATOMS):
                cps.append(
                    pltpu.make_async_copy(
                        ohs.at[sp, dd],
                        o_hbm.at[
                            pl.ds(
                                (a_ * _DEPTH + 2 * j2 + dd) * _PLANE_ROWS
                                + base,
                                _WROWS,
                            ),
                            :,
                        ],
                        sem,
                    )
                )
        return cps

    def start(cps):
        for cp in cps:
            cp.start()

    def wait(cps):
        for cp in cps:
            cp.wait()

    def sum5(bp, dd, r, cc):
        return (
            strips[bp, 0, dd, r, pl.ds(cc, 16)]
            + strips[bp, 1, dd, r, pl.ds(cc, 16)]
            + strips[bp, 2, dd, r, pl.ds(cc, 16)]
            + strips[bp, 3, dd, r, pl.ds(cc, 16)]
            + strips[bp, 4, dd, r, pl.ds(cc, 16)]
        )

    def compute_pair(idx, bp, d0, first):
        d0v = jnp.full((16,), d0, jnp.int32)

        def r_body(r, c):
            for k in range(8):
                cc = k * 16
                s0 = sum5(bp, 0, r, cc)
                s1 = sum5(bp, 1, r, cc)
                gt1 = s1 > s0
                sm = jnp.where(gt1, s1, s0)
                dm = jnp.where(gt1, d0v + 1, d0v)
                if first:
                    best[r, pl.ds(cc, 16)] = sm
                    idx[r, pl.ds(cc, 16)] = dm
                else:
                    b = best[r, pl.ds(cc, 16)]
                    gt = sm > b
                    best[r, pl.ds(cc, 16)] = jnp.where(gt, sm, b)
                    iv = idx[r, pl.ds(cc, 16)]
                    idx[r, pl.ds(cc, 16)] = jnp.where(gt, dm, iv)
            return c

        lax.fori_loop(0, _WROWS, r_body, 0)

    def p2_build(idx, j2, sp):
        d0v = jnp.full((16,), 2 * j2, jnp.int32)
        one = jnp.full((16,), 1.0, jnp.float32)
        zero = jnp.full((16,), 0.0, jnp.float32)

        def r_body(r, c):
            for k in range(8):
                cc = k * 16
                iv = idx[r, pl.ds(cc, 16)]
                ohs[sp, 0, r, pl.ds(cc, 16)] = jnp.where(iv == d0v, one, zero)
                ohs[sp, 1, r, pl.ds(cc, 16)] = jnp.where(
                    iv == d0v + 1, one, zero
                )
            return c

        lax.fori_loop(0, _WROWS, r_body, 0)

    sos = [so0, so1]
    base0 = wid * 64
    base1 = base0 + _WROWS

    start(in_pair(base0, 0, 0, si0))
    start(in_pair(base0, 2, 1, si1))
    wait(in_pair(base0, 0, 0, si0))
    compute_pair(idx0, 0, 0, True)
    start(in_pair(base0, 4, 0, si0))
    wait(in_pair(base0, 2, 1, si1))
    compute_pair(idx0, 1, jnp.int32(2), False)
    start(in_pair(base0, 6, 1, si1))

    def a_body(i, c):
        d0 = 4 * i + 4
        wait(in_pair(base0, d0, 0, si0))
        compute_pair(idx0, 0, d0, False)
        start(in_pair(base0, d0 + 4, 0, si0))
        wait(in_pair(base0, d0 + 2, 1, si1))
        compute_pair(idx0, 1, d0 + 2, False)

        @pl.when(d0 + 6 < _DEPTH)
        def _():
            start(in_pair(base0, d0 + 6, 1, si1))

        return c

    lax.fori_loop(0, 4, a_body, 0)
    start(in_pair(base1, 0, 1, si1))
    wait(in_pair(base0, _DEPTH - 2, 0, si0))
    compute_pair(idx0, 0, jnp.int32(_DEPTH - 2), False)

    start(in_pair(base1, 2, 0, si0))
    wait(in_pair(base1, 0, 1, si1))
    compute_pair(idx1, 1, 0, True)
    start(in_pair(base1, 4, 1, si1))
    p2_build(idx0, jnp.int32(0), 0)
    start(out_slot(base0, jnp.int32(0), 0, so0))
    wait(in_pair(base1, 2, 0, si0))
    compute_pair(idx1, 0, jnp.int32(2), False)
    start(in_pair(base1, 6, 0, si0))
    p2_build(idx0, jnp.int32(1), 1)
    start(out_slot(base0, jnp.int32(1), 1, so1))

    def b_body(i, c):
        d0 = 4 * i + 4
        j2a = 2 * i + 2
        wait(in_pair(base1, d0, 1, si1))
        compute_pair(idx1, 1, d0, False)
        start(in_pair(base1, d0 + 4, 1, si1))
        wait(out_slot(base0, j2a - 2, 0, so0))
        p2_build(idx0, j2a, 0)
        start(out_slot(base0, j2a, 0, so0))

        wait(in_pair(base1, d0 + 2, 0, si0))
        compute_pair(idx1, 0, d0 + 2, False)

        @pl.when(d0 + 6 < _DEPTH)
        def _():
            start(in_pair(base1, d0 + 6, 0, si0))

        wait(out_slot(base0, j2a - 1, 1, so1))
        p2_build(idx0, j2a + 1, 1)
        start(out_slot(base0, j2a + 1, 1, so1))
        return c

    lax.fori_loop(0, 4, b_body, 0)
    wait(in_pair(base1, _DEPTH - 2, 1, si1))
    compute_pair(idx1, 1, jnp.int32(_DEPTH - 2), False)
    wait(out_slot(base0, jnp.int32(8), 0, so0))
    p2_build(idx0, jnp.int32(10), 0)
    start(out_slot(base0, jnp.int32(10), 0, so0))

    wait(out_slot(base0, jnp.int32(10), 0, so0))
    p2_build(idx1, jnp.int32(0), 0)
    start(out_slot(base1, jnp.int32(0), 0, so0))
    wait(out_slot(base0, jnp.int32(9), 1, so1))
    p2_build(idx1, jnp.int32(1), 1)
    start(out_slot(base1, jnp.int32(1), 1, so1))

    def c_body(i, c):
        j2 = 2 * i + 2
        wait(out_slot(base1, j2 - 2, 0, so0))
        p2_build(idx1, j2, 0)
        start(out_slot(base1, j2, 0, so0))
        wait(out_slot(base1, j2 - 1, 1, so1))
        p2_build(idx1, j2 + 1, 1)
        start(out_slot(base1, j2 + 1, 1, so1))
        return c

    lax.fori_loop(0, 4, c_body, 0)
    wait(out_slot(base1, jnp.int32(8), 0, so0))
    p2_build(idx1, jnp.int32(10), 0)
    start(out_slot(base1, jnp.int32(10), 0, so0))
    wait(out_slot(base1, jnp.int32(9), 1, so1))
    wait(out_slot(base1, jnp.int32(10), 0, so0))


def kernel(inputs):
    b, l, a, d = inputs.shape
    x4 = (
        jnp.transpose(inputs, (2, 3, 0, 1))
        .reshape(_PLANES, b // 8, 8, l // 128, 128)
        .transpose(0, 1, 3, 2, 4)
        .reshape(a, d, _PLANE_ROWS, 128)
    )
    mesh = plsc.VectorSubcoreMesh(core_axis_name="c", subcore_axis_name="s")
    f = pl.kernel(
        _sc_body,
        out_type=jax.ShapeDtypeStruct((_ROWS, 128), jnp.float32),
        mesh=mesh,
        scratch_types=[
            pltpu.VMEM((2, _ATOMS, 2, _WROWS, 128), jnp.float32),
            pltpu.VMEM((_WROWS, 128), jnp.float32),
            pltpu.VMEM((_WROWS, 128), jnp.int32),
            pltpu.VMEM((_WROWS, 128), jnp.int32),
            pltpu.VMEM((2, 2, _WROWS, 128), jnp.float32),
            pltpu.SemaphoreType.DMA,
            pltpu.SemaphoreType.DMA,
            pltpu.SemaphoreType.DMA,
            pltpu.SemaphoreType.DMA,
        ],
    )
    o2 = f(x4)
    return (
        o2.reshape(_PLANES, b // 8, l // 128, 8, 128)
        .transpose(0, 1, 3, 2, 4)
        .reshape(a, d, b, l)
        .transpose(2, 3, 0, 1)
    )


# speedup vs baseline: 1.4421x; 1.0097x over previous
import jax
import jax.numpy as jnp
from jax import lax
from jax.experimental import pallas as pl
from jax.experimental.pallas import tpu as pltpu
from jax.experimental.pallas import tpu_sc as plsc

_DEPTH = 22
_ATOMS = 5
_PLANES = _ATOMS * _DEPTH
_PLANE_ROWS = 2048
_ROWS = _PLANES * _PLANE_ROWS
_WROWS = 32


def _sc_body(x4_hbm, o_hbm, strips, best, idx0, idx1, ohs, si0, si1, so0, so1):
    wid = lax.axis_index("s") * 2 + lax.axis_index("c")

    def in_pair(base, d0, bp, sem):
        return [
            pltpu.make_async_copy(
                x4_hbm.at[:, pl.ds(d0, 2), pl.ds(base, _WROWS), :],
                strips.at[bp],
                sem,
            )
        ]

    def out_slot(base, j2, sp, sem):
        cps = []
        for dd in range(2):
            for a_ in range(_ATOMS):
                cps.append(
                    pltpu.make_async_copy(
                        ohs.at[sp, dd],
                        o_hbm.at[
                            pl.ds(
                                (a_ * _DEPTH + 2 * j2 + dd) * _PLANE_ROWS
                                + base,
                                _WROWS,
                            ),
                            :,
                        ],
                        sem,
                    )
                )
        return cps

    def start(cps):
        for cp in cps:
            cp.start()

    def wait(cps):
        for cp in cps:
            cp.wait()

    def sum5(bp, dd, r, cc):
        return (
            strips[bp, 0, dd, r, pl.ds(cc, 16)]
            + strips[bp, 1, dd, r, pl.ds(cc, 16)]
            + strips[bp, 2, dd, r, pl.ds(cc, 16)]
            + strips[bp, 3, dd, r, pl.ds(cc, 16)]
            + strips[bp, 4, dd, r, pl.ds(cc, 16)]
        )

    def compute_pair(idx, bp, d0, first):
        d0v = jnp.full((16,), d0, jnp.int32)

        def r_body(r, c):
            for k in range(8):
                cc = k * 16
                s0 = sum5(bp, 0, r, cc)
                s1 = sum5(bp, 1, r, cc)
                gt1 = s1 > s0
                sm = jnp.where(gt1, s1, s0)
                dm = jnp.where(gt1, d0v + 1, d0v)
                if first:
                    best[r, pl.ds(cc, 16)] = sm
                    idx[r, pl.ds(cc, 16)] = dm
                else:
                    b = best[r, pl.ds(cc, 16)]
                    gt = sm > b
                    best[r, pl.ds(cc, 16)] = jnp.where(gt, sm, b)
                    iv = idx[r, pl.ds(cc, 16)]
                    idx[r, pl.ds(cc, 16)] = jnp.where(gt, dm, iv)
            return c

        lax.fori_loop(0, _WROWS, r_body, 0)

    def p2_build(idx, j2, sp):
        d0v = jnp.full((16,), 2 * j2, jnp.int32)
        one = jnp.full((16,), 1.0, jnp.float32)
        zero = jnp.full((16,), 0.0, jnp.float32)

        def r_body(r, c):
            for k in range(8):
                cc = k * 16
                iv = idx[r, pl.ds(cc, 16)]
                ohs[sp, 0, r, pl.ds(cc, 16)] = jnp.where(iv == d0v, one, zero)
                ohs[sp, 1, r, pl.ds(cc, 16)] = jnp.where(
                    iv == d0v + 1, one, zero
                )
            return c

        lax.fori_loop(0, _WROWS, r_body, 0)

    base0 = wid * 64
    base1 = base0 + _WROWS

    start(in_pair(base0, 0, 0, si0))
    start(in_pair(base0, 2, 1, si1))
    wait(in_pair(base0, 0, 0, si0))
    compute_pair(idx0, 0, 0, True)
    start(in_pair(base0, 4, 0, si0))
    wait(in_pair(base0, 2, 1, si1))
    compute_pair(idx0, 1, jnp.int32(2), False)
    start(in_pair(base0, 6, 1, si1))

    def a_body(i, c):
        d0 = 4 * i + 4
        wait(in_pair(base0, d0, 0, si0))
        compute_pair(idx0, 0, d0, False)
        start(in_pair(base0, d0 + 4, 0, si0))
        wait(in_pair(base0, d0 + 2, 1, si1))
        compute_pair(idx0, 1, d0 + 2, False)

        @pl.when(d0 + 6 < _DEPTH)
        def _():
            start(in_pair(base0, d0 + 6, 1, si1))

        return c

    lax.fori_loop(0, 4, a_body, 0)
    start(in_pair(base1, 0, 1, si1))
    wait(in_pair(base0, _DEPTH - 2, 0, si0))
    compute_pair(idx0, 0, jnp.int32(_DEPTH - 2), False)

    start(in_pair(base1, 2, 0, si0))
    wait(in_pair(base1, 0, 1, si1))
    compute_pair(idx1, 1, 0, True)
    start(in_pair(base1, 4, 1, si1))
    p2_build(idx0, jnp.int32(0), 0)
    start(out_slot(base0, jnp.int32(0), 0, so0))
    wait(in_pair(base1, 2, 0, si0))
    compute_pair(idx1, 0, jnp.int32(2), False)
    start(in_pair(base1, 6, 0, si0))
    p2_build(idx0, jnp.int32(1), 1)
    start(out_slot(base0, jnp.int32(1), 1, so1))

    def b_body(i, c):
        d0 = 4 * i + 4
        j2a = 2 * i + 2
        wait(in_pair(base1, d0, 1, si1))
        compute_pair(idx1, 1, d0, False)
        start(in_pair(base1, d0 + 4, 1, si1))
        wait(out_slot(base0, j2a - 2, 0, so0))
        p2_build(idx0, j2a, 0)
        start(out_slot(base0, j2a, 0, so0))

        wait(in_pair(base1, d0 + 2, 0, si0))
        compute_pair(idx1, 0, d0 + 2, False)

        @pl.when(d0 + 6 < _DEPTH)
        def _():
            start(in_pair(base1, d0 + 6, 0, si0))

        wait(out_slot(base0, j2a - 1, 1, so1))
        p2_build(idx0, j2a + 1, 1)
        start(out_slot(base0, j2a + 1, 1, so1))
        return c

    lax.fori_loop(0, 4, b_body, 0)
    wait(in_pair(base1, _DEPTH - 2, 1, si1))
    compute_pair(idx1, 1, jnp.int32(_DEPTH - 2), False)
    wait(out_slot(base0, jnp.int32(8), 0, so0))
    p2_build(idx0, jnp.int32(10), 0)
    start(out_slot(base0, jnp.int32(10), 0, so0))

    wait(out_slot(base0, jnp.int32(10), 0, so0))
    p2_build(idx1, jnp.int32(0), 0)
    start(out_slot(base1, jnp.int32(0), 0, so0))
    wait(out_slot(base0, jnp.int32(9), 1, so1))
    p2_build(idx1, jnp.int32(1), 1)
    start(out_slot(base1, jnp.int32(1), 1, so1))

    def c_body(i, c):
        j2 = 2 * i + 2
        wait(out_slot(base1, j2 - 2, 0, so0))
        p2_build(idx1, j2, 0)
        start(out_slot(base1, j2, 0, so0))
        wait(out_slot(base1, j2 - 1, 1, so1))
        p2_build(idx1, j2 + 1, 1)
        start(out_slot(base1, j2 + 1, 1, so1))
        return c

    lax.fori_loop(0, 4, c_body, 0)
    wait(out_slot(base1, jnp.int32(8), 0, so0))
    p2_build(idx1, jnp.int32(10), 0)
    start(out_slot(base1, jnp.int32(10), 0, so0))
    wait(out_slot(base1, jnp.int32(9), 1, so1))
    wait(out_slot(base1, jnp.int32(10), 0, so0))


def kernel(inputs):
    b, l, a, d = inputs.shape
    x4 = (
        jnp.transpose(inputs, (2, 3, 0, 1))
        .reshape(_PLANES, b // 8, 8, l // 128, 128)
        .transpose(0, 1, 3, 2, 4)
        .reshape(a, d, _PLANE_ROWS, 128)
    )
    mesh = plsc.VectorSubcoreMesh(core_axis_name="c", subcore_axis_name="s")
    f = pl.kernel(
        _sc_body,
        out_type=jax.ShapeDtypeStruct((_ROWS, 128), jnp.float32),
        mesh=mesh,
        scratch_types=[
            pltpu.VMEM((2, _ATOMS, 2, _WROWS, 128), jnp.float32),
            pltpu.VMEM((_WROWS, 128), jnp.float32),
            pltpu.VMEM((_WROWS, 128), jnp.int32),
            pltpu.VMEM((_WROWS, 128), jnp.int32),
            pltpu.VMEM((2, 2, _WROWS, 128), jnp.float32),
            pltpu.SemaphoreType.DMA,
            pltpu.SemaphoreType.DMA,
            pltpu.SemaphoreType.DMA,
            pltpu.SemaphoreType.DMA,
        ],
    )
    o2 = f(x4)
    return (
        o2.reshape(_PLANES, b // 8, l // 128, 8, 128)
        .transpose(0, 1, 3, 2, 4)
        .reshape(a, d, b, l)
        .transpose(2, 3, 0, 1)
    )
